# baseline (speedup 1.0000x reference)
import functools

import jax
import jax.numpy as jnp
from jax import lax
from jax.experimental import pallas as pl
from jax.experimental.pallas import tpu as pltpu
from jax.experimental.pallas import tpu_sc as plsc

_K = 16
_CAND = 32
_BQ = 128
_M = 4096
_N = 16384
_NWORK = 32
_ROWS_W = _M // _NWORK


def _lex(av, ai, bv, bi):
    return av < bv + jnp.where(ai < bi, jnp.int32(1), jnp.int32(0))


def _ce(av, ai, bv, bi):
    c = _lex(av, ai, bv, bi)
    return (jnp.where(c, av, bv), jnp.where(c, ai, bi),
            jnp.where(c, bv, av), jnp.where(c, bi, ai))


def _bitonic_clean(vals, idxs):
    s = len(vals)
    if s == 1:
        return vals, idxs
    half = s // 2
    vals = list(vals)
    idxs = list(idxs)
    for j in range(half):
        lo_v, lo_i, hi_v, hi_i = _ce(vals[j], idxs[j],
                                     vals[j + half], idxs[j + half])
        vals[j], vals[j + half] = lo_v, hi_v
        idxs[j], idxs[j + half] = lo_i, hi_i
    lv, li = _bitonic_clean(vals[:half], idxs[:half])
    hv, hi = _bitonic_clean(vals[half:], idxs[half:])
    return lv + hv, li + hi


def _merge(av, ai, bv, bi):
    s = len(av)
    if 2 * s <= _K:
        xv = list(av) + list(bv[::-1])
        xi = list(ai) + list(bi[::-1])
        return _bitonic_clean(xv, xi)
    lv, li = [], []
    for j in range(s):
        c = _lex(av[j], ai[j], bv[s - 1 - j], bi[s - 1 - j])
        lv.append(jnp.where(c, av[j], bv[s - 1 - j]))
        li.append(jnp.where(c, ai[j], bi[s - 1 - j]))
    return _bitonic_clean(lv, li)


def _oddeven_merge(lo, hi, r, pairs):
    step = r * 2
    if step < hi - lo:
        _oddeven_merge(lo, hi, step, pairs)
        _oddeven_merge(lo + r, hi, step, pairs)
        for i in range(lo + r, hi - r, step):
            pairs.append((i, i + r))
    else:
        pairs.append((lo, lo + r))


def _oddeven_sort_pairs(lo, hi, pairs):
    if hi - lo >= 1:
        mid = lo + (hi - lo) // 2
        _oddeven_sort_pairs(lo, mid, pairs)
        _oddeven_sort_pairs(mid + 1, hi, pairs)
        _oddeven_merge(lo, hi, 1, pairs)


_SORT16_PAIRS = []
_oddeven_sort_pairs(0, 15, _SORT16_PAIRS)


def _knn_block(q_ref, bt_ref, cv_ref, ci_ref):
    q = q_ref[...]
    bt = bt_ref[...]
    q2 = jnp.sum(q * q, axis=1, keepdims=True)
    b2 = jnp.sum(bt * bt, axis=0, keepdims=True)
    qb = jax.lax.dot_general(
        q.astype(jnp.bfloat16), bt.astype(jnp.bfloat16),
        dimension_numbers=(((1,), (0,)), ((), ())),
        preferred_element_type=jnp.float32)
    d2 = q2 - 2.0 * qb + b2
    iota = jax.lax.broadcasted_iota(jnp.int32, d2.shape, 1)
    bits = jax.lax.bitcast_convert_type(d2, jnp.int32)
    d2 = jnp.where(bits < 0, bits ^ jnp.int32(0x7FFFFFFF), bits)

    n16 = d2.shape[1] // _K
    vals = [d2[:, j * n16:(j + 1) * n16] for j in range(_K)]
    idxs = [iota[:, j * n16:(j + 1) * n16] for j in range(_K)]
    for a, b in _SORT16_PAIRS:
        lo_v, lo_i, hi_v, hi_i = _ce(vals[a], idxs[a], vals[b], idxs[b])
        vals[a], vals[b] = lo_v, hi_v
        idxs[a], idxs[b] = lo_i, hi_i
    width = n16
    while width > 2:
        half = width // 2
        a_v = [p[:, :half] for p in vals]
        b_v = [p[:, half:] for p in vals]
        a_i = [p[:, :half] for p in idxs]
        b_i = [p[:, half:] for p in idxs]
        vals, idxs = _merge(a_v, a_i, b_v, b_i)
        width = half

    kcat = jnp.concatenate(vals, axis=1)
    kcat = jnp.where(kcat < 0, kcat ^ jnp.int32(0x7FFFFFFF), kcat)
    cv_ref[...] = jax.lax.bitcast_convert_type(kcat, jnp.float32)
    ci_ref[...] = jnp.concatenate(idxs, axis=1)


@jax.jit
def _knn_candidates(qp, btp):
    return pl.pallas_call(
        _knn_block,
        grid=(_M // _BQ,),
        in_specs=[
            pl.BlockSpec((_BQ, 8), lambda i: (i, 0)),
            pl.BlockSpec((8, _N), lambda i: (0, 0)),
        ],
        out_specs=[
            pl.BlockSpec((_BQ, _CAND), lambda i: (i, 0)),
            pl.BlockSpec((_BQ, _CAND), lambda i: (i, 0)),
        ],
        out_shape=[
            jax.ShapeDtypeStruct((_M, _CAND), jnp.float32),
            jax.ShapeDtypeStruct((_M, _CAND), jnp.int32),
        ],
        compiler_params=pltpu.CompilerParams(
            dimension_semantics=("parallel",)),
    )(qp, btp)


def _sc_select_body(cv_hbm, ci_hbm, dv_hbm, di_hbm, cv_v, ci_v, dv_v, di_v):
    wid = lax.axis_index("s") * 2 + lax.axis_index("c")
    base = wid * _ROWS_W
    pltpu.sync_copy(cv_hbm.at[pl.ds(base, _ROWS_W)], cv_v)
    pltpu.sync_copy(ci_hbm.at[pl.ds(base, _ROWS_W)], ci_v)
    one = jnp.full((16,), 1, jnp.int32)
    zero = jnp.full((16,), 0, jnp.int32)
    lane = lax.broadcasted_iota(jnp.int32, (16,), 0)

    def row_body(r, carry):
        v0 = cv_v[r, pl.ds(0, 16)]
        v1 = cv_v[r, pl.ds(16, 16)]
        i0 = ci_v[r, pl.ds(0, 16)]
        i1 = ci_v[r, pl.ds(16, 16)]
        r0 = zero
        r1 = zero
        for c in range(_CAND):
            vh, ih = (v0, i0) if c < 16 else (v1, i1)
            sv = jnp.full((16,), vh[c % 16], jnp.float32)
            si = jnp.full((16,), ih[c % 16], jnp.int32)
            lt0 = (sv < v0) | ((sv == v0) & (si < i0))
            lt1 = (sv < v1) | ((sv == v1) & (si < i1))
            r0 = r0 + jnp.where(lt0, one, zero)
            r1 = r1 + jnp.where(lt1, one, zero)
        ov = jnp.full((16,), 0.0, jnp.float32)
        oi = zero
        for c in range(_CAND):
            vh, ih, rh = (v0, i0, r0) if c < 16 else (v1, i1, r1)
            hit = jnp.full((16,), rh[c % 16], jnp.int32) == lane
            ov = jnp.where(hit, jnp.full((16,), vh[c % 16], jnp.float32), ov)
            oi = jnp.where(hit, jnp.full((16,), ih[c % 16], jnp.int32), oi)
        dv_v[pl.ds(r * _K, _K)] = ov
        di_v[pl.ds(r * _K, _K)] = oi
        return carry

    lax.fori_loop(0, _ROWS_W, row_body, 0)
    pltpu.sync_copy(dv_v, dv_hbm.at[pl.ds(base * _K, _ROWS_W * _K)])
    pltpu.sync_copy(di_v, di_hbm.at[pl.ds(base * _K, _ROWS_W * _K)])


@jax.jit
def _sc_select(cv, ci):
    return pl.kernel(
        _sc_select_body,
        mesh=plsc.VectorSubcoreMesh(core_axis_name="c", subcore_axis_name="s"),
        out_type=[
            jax.ShapeDtypeStruct((_M * _K,), jnp.float32),
            jax.ShapeDtypeStruct((_M * _K,), jnp.int32),
        ],
        scratch_types=[
            pltpu.VMEM((_ROWS_W, _CAND), jnp.float32),
            pltpu.VMEM((_ROWS_W, _CAND), jnp.int32),
            pltpu.VMEM((_ROWS_W * _K,), jnp.float32),
            pltpu.VMEM((_ROWS_W * _K,), jnp.int32),
        ],
    )(cv, ci)


def kernel(queries, base, k):
    del k
    qp = jnp.pad(queries, ((0, 0), (0, 5)))
    btp = jnp.pad(base, ((0, 0), (0, 5))).T
    cv, ci = _knn_candidates(qp, btp)
    dists, idx = _sc_select(cv, ci)
    return dists.reshape(_M, _K), idx.reshape(_M, _K)


# speedup vs baseline: 1.6039x; 1.6039x over previous
import functools

import jax
import jax.numpy as jnp
from jax import lax
from jax.experimental import pallas as pl
from jax.experimental.pallas import tpu as pltpu
from jax.experimental.pallas import tpu_sc as plsc

_K = 16
_CAND = 32
_BQ = 128
_M = 4096
_N = 16384
_NWORK = 32
_ROWS_W = _M // _NWORK


def _lex(av, ai, bv, bi):
    return (av < bv) | ((av == bv) & (ai < bi))


def _ce(av, ai, bv, bi):
    c = _lex(av, ai, bv, bi)
    return (jnp.where(c, av, bv), jnp.where(c, ai, bi),
            jnp.where(c, bv, av), jnp.where(c, bi, ai))


def _bitonic_clean(vals, idxs):
    s = len(vals)
    if s == 1:
        return vals, idxs
    half = s // 2
    vals = list(vals)
    idxs = list(idxs)
    for j in range(half):
        lo_v, lo_i, hi_v, hi_i = _ce(vals[j], idxs[j],
                                     vals[j + half], idxs[j + half])
        vals[j], vals[j + half] = lo_v, hi_v
        idxs[j], idxs[j + half] = lo_i, hi_i
    lv, li = _bitonic_clean(vals[:half], idxs[:half])
    hv, hi = _bitonic_clean(vals[half:], idxs[half:])
    return lv + hv, li + hi


def _merge(av, ai, bv, bi):
    s = len(av)
    if 2 * s <= _K:
        xv = list(av) + list(bv[::-1])
        xi = list(ai) + list(bi[::-1])
        return _bitonic_clean(xv, xi)
    lv, li = [], []
    for j in range(s):
        c = _lex(av[j], ai[j], bv[s - 1 - j], bi[s - 1 - j])
        lv.append(jnp.where(c, av[j], bv[s - 1 - j]))
        li.append(jnp.where(c, ai[j], bi[s - 1 - j]))
    return _bitonic_clean(lv, li)


def _oddeven_merge(lo, hi, r, pairs):
    step = r * 2
    if step < hi - lo:
        _oddeven_merge(lo, hi, step, pairs)
        _oddeven_merge(lo + r, hi, step, pairs)
        for i in range(lo + r, hi - r, step):
            pairs.append((i, i + r))
    else:
        pairs.append((lo, lo + r))


def _oddeven_sort_pairs(lo, hi, pairs):
    if hi - lo >= 1:
        mid = lo + (hi - lo) // 2
        _oddeven_sort_pairs(lo, mid, pairs)
        _oddeven_sort_pairs(mid + 1, hi, pairs)
        _oddeven_merge(lo, hi, 1, pairs)


_SORT16_PAIRS = []
_oddeven_sort_pairs(0, 15, _SORT16_PAIRS)


def _knn_block(q_ref, bt_ref, cv_ref, ci_ref):
    q = q_ref[...]
    bt = bt_ref[...]
    q2 = jnp.sum(q * q, axis=1, keepdims=True)
    b2 = jnp.sum(bt * bt, axis=0, keepdims=True)
    qb = jax.lax.dot_general(
        q.astype(jnp.bfloat16), bt.astype(jnp.bfloat16),
        dimension_numbers=(((1,), (0,)), ((), ())),
        preferred_element_type=jnp.float32)
    d2 = q2 - 2.0 * qb + b2
    iota = jax.lax.broadcasted_iota(jnp.int32, d2.shape, 1)

    n16 = d2.shape[1] // _K
    vals = [d2[:, j * n16:(j + 1) * n16] for j in range(_K)]
    idxs = [iota[:, j * n16:(j + 1) * n16] for j in range(_K)]
    for a, b in _SORT16_PAIRS:
        lo_v, lo_i, hi_v, hi_i = _ce(vals[a], idxs[a], vals[b], idxs[b])
        vals[a], vals[b] = lo_v, hi_v
        idxs[a], idxs[b] = lo_i, hi_i
    width = n16
    transposed = False
    while width > 2:
        if width == 128 and not transposed:
            vals = [p.T for p in vals]
            idxs = [p.T for p in idxs]
            transposed = True
        half = width // 2
        if transposed:
            a_v = [p[:half, :] for p in vals]
            b_v = [p[half:, :] for p in vals]
            a_i = [p[:half, :] for p in idxs]
            b_i = [p[half:, :] for p in idxs]
        else:
            a_v = [p[:, :half] for p in vals]
            b_v = [p[:, half:] for p in vals]
            a_i = [p[:, :half] for p in idxs]
            b_i = [p[:, half:] for p in idxs]
        vals, idxs = _merge(a_v, a_i, b_v, b_i)
        width = half

    cv_ref[...] = jnp.concatenate(vals, axis=0).T
    ci_ref[...] = jnp.concatenate(idxs, axis=0).T


@jax.jit
def _knn_candidates(qp, btp):
    return pl.pallas_call(
        _knn_block,
        grid=(_M // _BQ,),
        in_specs=[
            pl.BlockSpec((_BQ, 8), lambda i: (i, 0)),
            pl.BlockSpec((8, _N), lambda i: (0, 0)),
        ],
        out_specs=[
            pl.BlockSpec((_BQ, _CAND), lambda i: (i, 0)),
            pl.BlockSpec((_BQ, _CAND), lambda i: (i, 0)),
        ],
        out_shape=[
            jax.ShapeDtypeStruct((_M, _CAND), jnp.float32),
            jax.ShapeDtypeStruct((_M, _CAND), jnp.int32),
        ],
        compiler_params=pltpu.CompilerParams(
            dimension_semantics=("parallel",)),
    )(qp, btp)


def _sc_select_body(cv_hbm, ci_hbm, dv_hbm, di_hbm, cv_v, ci_v, dv_v, di_v):
    wid = lax.axis_index("s") * 2 + lax.axis_index("c")
    base = wid * _ROWS_W
    pltpu.sync_copy(cv_hbm.at[pl.ds(base, _ROWS_W)], cv_v)
    pltpu.sync_copy(ci_hbm.at[pl.ds(base, _ROWS_W)], ci_v)
    one = jnp.full((16,), 1, jnp.int32)
    zero = jnp.full((16,), 0, jnp.int32)
    lane = lax.broadcasted_iota(jnp.int32, (16,), 0)

    def row_body(r, carry):
        v0 = cv_v[r, pl.ds(0, 16)]
        v1 = cv_v[r, pl.ds(16, 16)]
        i0 = ci_v[r, pl.ds(0, 16)]
        i1 = ci_v[r, pl.ds(16, 16)]
        r0 = zero
        r1 = zero
        for c in range(_CAND):
            vh, ih = (v0, i0) if c < 16 else (v1, i1)
            sv = jnp.full((16,), vh[c % 16], jnp.float32)
            si = jnp.full((16,), ih[c % 16], jnp.int32)
            lt0 = (sv < v0) | ((sv == v0) & (si < i0))
            lt1 = (sv < v1) | ((sv == v1) & (si < i1))
            r0 = r0 + jnp.where(lt0, one, zero)
            r1 = r1 + jnp.where(lt1, one, zero)
        ov = jnp.full((16,), 0.0, jnp.float32)
        oi = zero
        for c in range(_CAND):
            vh, ih, rh = (v0, i0, r0) if c < 16 else (v1, i1, r1)
            hit = jnp.full((16,), rh[c % 16], jnp.int32) == lane
            ov = jnp.where(hit, jnp.full((16,), vh[c % 16], jnp.float32), ov)
            oi = jnp.where(hit, jnp.full((16,), ih[c % 16], jnp.int32), oi)
        dv_v[pl.ds(r * _K, _K)] = ov
        di_v[pl.ds(r * _K, _K)] = oi
        return carry

    lax.fori_loop(0, _ROWS_W, row_body, 0)
    pltpu.sync_copy(dv_v, dv_hbm.at[pl.ds(base * _K, _ROWS_W * _K)])
    pltpu.sync_copy(di_v, di_hbm.at[pl.ds(base * _K, _ROWS_W * _K)])


@jax.jit
def _sc_select(cv, ci):
    return pl.kernel(
        _sc_select_body,
        mesh=plsc.VectorSubcoreMesh(core_axis_name="c", subcore_axis_name="s"),
        out_type=[
            jax.ShapeDtypeStruct((_M * _K,), jnp.float32),
            jax.ShapeDtypeStruct((_M * _K,), jnp.int32),
        ],
        scratch_types=[
            pltpu.VMEM((_ROWS_W, _CAND), jnp.float32),
            pltpu.VMEM((_ROWS_W, _CAND), jnp.int32),
            pltpu.VMEM((_ROWS_W * _K,), jnp.float32),
            pltpu.VMEM((_ROWS_W * _K,), jnp.int32),
        ],
    )(cv, ci)


def kernel(queries, base, k):
    del k
    qp = jnp.pad(queries, ((0, 0), (0, 5)))
    btp = jnp.pad(base, ((0, 0), (0, 5))).T
    cv, ci = _knn_candidates(qp, btp)
    dists, idx = _sc_select(cv, ci)
    return dists.reshape(_M, _K), idx.reshape(_M, _K)


# speedup vs baseline: 1.6575x; 1.0334x over previous
import functools

import jax
import jax.numpy as jnp
from jax import lax
from jax.experimental import pallas as pl
from jax.experimental.pallas import tpu as pltpu
from jax.experimental.pallas import tpu_sc as plsc

_K = 16
_CAND = 32
_BQ = 128
_M = 4096
_N = 16384
_NWORK = 32
_ROWS_W = _M // _NWORK


def _lex(av, ai, bv, bi):
    return (av < bv) | ((av == bv) & (ai < bi))


def _ce(av, ai, bv, bi):
    c = _lex(av, ai, bv, bi)
    return (jnp.where(c, av, bv), jnp.where(c, ai, bi),
            jnp.where(c, bv, av), jnp.where(c, bi, ai))


def _bitonic_clean(vals, idxs):
    s = len(vals)
    if s == 1:
        return vals, idxs
    half = s // 2
    vals = list(vals)
    idxs = list(idxs)
    for j in range(half):
        lo_v, lo_i, hi_v, hi_i = _ce(vals[j], idxs[j],
                                     vals[j + half], idxs[j + half])
        vals[j], vals[j + half] = lo_v, hi_v
        idxs[j], idxs[j + half] = lo_i, hi_i
    lv, li = _bitonic_clean(vals[:half], idxs[:half])
    hv, hi = _bitonic_clean(vals[half:], idxs[half:])
    return lv + hv, li + hi


def _merge(av, ai, bv, bi):
    s = len(av)
    if 2 * s <= _K:
        xv = list(av) + list(bv[::-1])
        xi = list(ai) + list(bi[::-1])
        return _bitonic_clean(xv, xi)
    lv, li = [], []
    for j in range(s):
        c = _lex(av[j], ai[j], bv[s - 1 - j], bi[s - 1 - j])
        lv.append(jnp.where(c, av[j], bv[s - 1 - j]))
        li.append(jnp.where(c, ai[j], bi[s - 1 - j]))
    return _bitonic_clean(lv, li)


def _oddeven_merge(lo, hi, r, pairs):
    step = r * 2
    if step < hi - lo:
        _oddeven_merge(lo, hi, step, pairs)
        _oddeven_merge(lo + r, hi, step, pairs)
        for i in range(lo + r, hi - r, step):
            pairs.append((i, i + r))
    else:
        pairs.append((lo, lo + r))


def _oddeven_sort_pairs(lo, hi, pairs):
    if hi - lo >= 1:
        mid = lo + (hi - lo) // 2
        _oddeven_sort_pairs(lo, mid, pairs)
        _oddeven_sort_pairs(mid + 1, hi, pairs)
        _oddeven_merge(lo, hi, 1, pairs)


_SORT16_PAIRS = []
_oddeven_sort_pairs(0, 15, _SORT16_PAIRS)


def _knn_block(q_ref, bt_ref, cv_ref, ci_ref):
    q = q_ref[...]
    bt = bt_ref[...]
    q2 = jnp.sum(q * q, axis=1, keepdims=True)
    b2 = jnp.sum(bt * bt, axis=0, keepdims=True)
    qb = jax.lax.dot_general(
        q.astype(jnp.bfloat16), bt.astype(jnp.bfloat16),
        dimension_numbers=(((1,), (0,)), ((), ())),
        preferred_element_type=jnp.float32)
    d2 = q2 - 2.0 * qb + b2
    iota = jax.lax.broadcasted_iota(jnp.int32, d2.shape, 1)

    n16 = d2.shape[1] // _K
    vals = [d2[:, j * n16:(j + 1) * n16] for j in range(_K)]
    idxs = [iota[:, j * n16:(j + 1) * n16] for j in range(_K)]
    virgin = set(range(_K))
    for a, b in _SORT16_PAIRS:
        if a in virgin and b in virgin:
            c = vals[a] <= vals[b]
            lo_v, hi_v = jnp.where(c, vals[a], vals[b]), jnp.where(c, vals[b], vals[a])
            lo_i, hi_i = jnp.where(c, idxs[a], idxs[b]), jnp.where(c, idxs[b], idxs[a])
        else:
            lo_v, lo_i, hi_v, hi_i = _ce(vals[a], idxs[a], vals[b], idxs[b])
        virgin.discard(a)
        virgin.discard(b)
        vals[a], vals[b] = lo_v, hi_v
        idxs[a], idxs[b] = lo_i, hi_i
    width = n16
    transposed = False
    while width > 2:
        if width == 128 and not transposed:
            vals = [p.T for p in vals]
            idxs = [p.T for p in idxs]
            transposed = True
        half = width // 2
        if transposed:
            a_v = [p[:half, :] for p in vals]
            b_v = [p[half:, :] for p in vals]
            a_i = [p[:half, :] for p in idxs]
            b_i = [p[half:, :] for p in idxs]
        else:
            a_v = [p[:, :half] for p in vals]
            b_v = [p[:, half:] for p in vals]
            a_i = [p[:, :half] for p in idxs]
            b_i = [p[:, half:] for p in idxs]
        vals, idxs = _merge(a_v, a_i, b_v, b_i)
        width = half

    cv_ref[...] = jnp.concatenate(vals, axis=0).T
    ci_ref[...] = jnp.concatenate(idxs, axis=0).T


@jax.jit
def _knn_candidates(qp, btp):
    return pl.pallas_call(
        _knn_block,
        grid=(_M // _BQ,),
        in_specs=[
            pl.BlockSpec((_BQ, 8), lambda i: (i, 0)),
            pl.BlockSpec((8, _N), lambda i: (0, 0)),
        ],
        out_specs=[
            pl.BlockSpec((_BQ, _CAND), lambda i: (i, 0)),
            pl.BlockSpec((_BQ, _CAND), lambda i: (i, 0)),
        ],
        out_shape=[
            jax.ShapeDtypeStruct((_M, _CAND), jnp.float32),
            jax.ShapeDtypeStruct((_M, _CAND), jnp.int32),
        ],
        compiler_params=pltpu.CompilerParams(
            dimension_semantics=("parallel",)),
    )(qp, btp)


def _sc_select_body(cv_hbm, ci_hbm, dv_hbm, di_hbm, cv_v, ci_v, dv_v, di_v):
    wid = lax.axis_index("s") * 2 + lax.axis_index("c")
    base = wid * _ROWS_W
    pltpu.sync_copy(cv_hbm.at[pl.ds(base, _ROWS_W)], cv_v)
    pltpu.sync_copy(ci_hbm.at[pl.ds(base, _ROWS_W)], ci_v)
    one = jnp.full((16,), 1, jnp.int32)
    zero = jnp.full((16,), 0, jnp.int32)
    lane = lax.broadcasted_iota(jnp.int32, (16,), 0)

    def row_body(r, carry):
        v0 = cv_v[r, pl.ds(0, 16)]
        v1 = cv_v[r, pl.ds(16, 16)]
        i0 = ci_v[r, pl.ds(0, 16)]
        i1 = ci_v[r, pl.ds(16, 16)]
        r0 = zero
        r1 = zero
        for c in range(_CAND):
            vh, ih = (v0, i0) if c < 16 else (v1, i1)
            sv = jnp.full((16,), vh[c % 16], jnp.float32)
            si = jnp.full((16,), ih[c % 16], jnp.int32)
            lt0 = (sv < v0) | ((sv == v0) & (si < i0))
            lt1 = (sv < v1) | ((sv == v1) & (si < i1))
            r0 = r0 + jnp.where(lt0, one, zero)
            r1 = r1 + jnp.where(lt1, one, zero)
        ov = jnp.full((16,), 0.0, jnp.float32)
        oi = zero
        for c in range(_CAND):
            vh, ih, rh = (v0, i0, r0) if c < 16 else (v1, i1, r1)
            hit = jnp.full((16,), rh[c % 16], jnp.int32) == lane
            ov = jnp.where(hit, jnp.full((16,), vh[c % 16], jnp.float32), ov)
            oi = jnp.where(hit, jnp.full((16,), ih[c % 16], jnp.int32), oi)
        dv_v[pl.ds(r * _K, _K)] = ov
        di_v[pl.ds(r * _K, _K)] = oi
        return carry

    lax.fori_loop(0, _ROWS_W, row_body, 0)
    pltpu.sync_copy(dv_v, dv_hbm.at[pl.ds(base * _K, _ROWS_W * _K)])
    pltpu.sync_copy(di_v, di_hbm.at[pl.ds(base * _K, _ROWS_W * _K)])


@jax.jit
def _sc_select(cv, ci):
    return pl.kernel(
        _sc_select_body,
        mesh=plsc.VectorSubcoreMesh(core_axis_name="c", subcore_axis_name="s"),
        out_type=[
            jax.ShapeDtypeStruct((_M * _K,), jnp.float32),
            jax.ShapeDtypeStruct((_M * _K,), jnp.int32),
        ],
        scratch_types=[
            pltpu.VMEM((_ROWS_W, _CAND), jnp.float32),
            pltpu.VMEM((_ROWS_W, _CAND), jnp.int32),
            pltpu.VMEM((_ROWS_W * _K,), jnp.float32),
            pltpu.VMEM((_ROWS_W * _K,), jnp.int32),
        ],
    )(cv, ci)


def kernel(queries, base, k):
    del k
    qp = jnp.pad(queries, ((0, 0), (0, 5)))
    btp = jnp.pad(base, ((0, 0), (0, 5))).T
    cv, ci = _knn_candidates(qp, btp)
    dists, idx = _sc_select(cv, ci)
    return dists.reshape(_M, _K), idx.reshape(_M, _K)
